# Initial kernel scaffold; baseline (speedup 1.0000x reference)
import jax
import jax.numpy as jnp
from jax.experimental import pallas as pl


def kernel(x, edge_index, batch, W1, b1, W2, b2, gamma, beta, Wm1, bm1, Wm2, bm2):
    raise NotImplementedError("write your pallas kernel here")


# speedup vs baseline: 9.6649x; 9.6649x over previous
import dataclasses

import jax
import jax.numpy as jnp
from jax import lax
from jax.experimental import pallas as pl
from jax.experimental.pallas import tpu as pltpu
from jax.experimental.pallas import tpu_sc as plsc

_SC_PARAMS = pltpu.CompilerParams()
if "needs_layout_passes" in pltpu.CompilerParams.__dataclass_fields__:
    _SC_PARAMS = dataclasses.replace(_SC_PARAMS, needs_layout_passes=False)

_N = 10000
_E = 320000
_D = 128
_H = 128
_G = 64

_NC = 2
_NS = 16
_NW = _NC * _NS
_CHUNK = 128
_NCHUNK = 79
_EPT = _CHUNK * _NCHUNK
_EPAD = _EPT * _NW
_NR = 10240
_RPT = _NR // _NS
_EPW = _E // _NW

_BLK = 400
_NBLK = _N // _BLK


def _deg_body(dst_hbm, ones_hbm, out_hbm, dbuf, onesv, zb, sh):
    cid = lax.axis_index("c")
    sid = lax.axis_index("s")
    wid = cid * _NS + sid

    @pl.loop(0, 16)
    def _(r):
        @pl.loop(0, 8)
        def _(c):
            zb[r, pl.ds(c * 16, 16)] = jnp.zeros((16,), jnp.float32)

    @pl.loop(0, _RPT // 16)
    def _(t):
        pltpu.sync_copy(zb, sh.at[pl.ds(sid * _RPT + t * 16, 16)])

    pltpu.sync_copy(ones_hbm, onesv)
    plsc.subcore_barrier()

    @pl.loop(0, _NCHUNK)
    def _(c):
        row = wid * _NCHUNK + c
        pltpu.sync_copy(dst_hbm.at[pl.ds(row, 1)], dbuf)
        pltpu.sync_copy(onesv, sh.at[dbuf.at[0]], add=True)

    plsc.subcore_barrier()
    pltpu.sync_copy(sh.at[pl.ds(sid * _RPT, _RPT)],
                    out_hbm.at[cid].at[pl.ds(sid * _RPT, _RPT)])


def _deg(dst2d, ones):
    mesh = plsc.VectorSubcoreMesh(core_axis_name="c", subcore_axis_name="s")
    k = pl.kernel(
        _deg_body,
        out_type=jax.ShapeDtypeStruct((_NC, _NR, _H), jnp.float32),
        mesh=mesh,
        scratch_types=[
            pltpu.VMEM((1, _CHUNK), jnp.int32),
            pltpu.VMEM((_CHUNK, _H), jnp.float32),
            pltpu.VMEM((16, _H), jnp.float32),
            pltpu.VMEM_SHARED((_NR, _H), jnp.float32),
        ],
        compiler_params=_SC_PARAMS,
    )
    return k(dst2d, ones)


def _agg_body(tab_hbm, src_hbm, dst_hbm, out_hbm, sbuf, dbuf, rows, zb, acc):
    cid = lax.axis_index("c")
    sid = lax.axis_index("s")
    wid = cid * _NS + sid

    @pl.loop(0, 16)
    def _(r):
        @pl.loop(0, 8)
        def _(c):
            zb[r, pl.ds(c * 16, 16)] = jnp.zeros((16,), jnp.float32)

    @pl.loop(0, _RPT // 16)
    def _(t):
        pltpu.sync_copy(zb, acc.at[pl.ds(sid * _RPT + t * 16, 16)])

    plsc.subcore_barrier()

    @pl.loop(0, _NCHUNK)
    def _(c):
        row = wid * _NCHUNK + c
        pltpu.sync_copy(src_hbm.at[pl.ds(row, 1)], sbuf)
        pltpu.sync_copy(dst_hbm.at[pl.ds(row, 1)], dbuf)
        pltpu.sync_copy(tab_hbm.at[sbuf.at[0]], rows)
        pltpu.sync_copy(rows, acc.at[dbuf.at[0]], add=True)

    plsc.subcore_barrier()
    pltpu.sync_copy(acc.at[pl.ds(sid * _RPT, _RPT)],
                    out_hbm.at[cid].at[pl.ds(sid * _RPT, _RPT)])


def _agg(tab, src2d, dst2d):
    mesh = plsc.VectorSubcoreMesh(core_axis_name="c", subcore_axis_name="s")
    k = pl.kernel(
        _agg_body,
        out_type=jax.ShapeDtypeStruct((_NC, _NR, _H), jnp.float32),
        mesh=mesh,
        scratch_types=[
            pltpu.VMEM((1, _CHUNK), jnp.int32),
            pltpu.VMEM((1, _CHUNK), jnp.int32),
            pltpu.VMEM((_CHUNK, _H), jnp.float32),
            pltpu.VMEM((16, _H), jnp.float32),
            pltpu.VMEM_SHARED((_NR, _H), jnp.float32),
        ],
        compiler_params=_SC_PARAMS,
    )
    return k(tab, src2d, dst2d)


def _mm_body(x_ref, w_ref, o_ref):
    o_ref[...] = lax.dot_general(
        x_ref[...], w_ref[...], (((1,), (0,)), ((), ())),
        preferred_element_type=jnp.float32,
        precision=lax.Precision.HIGHEST)


def _mm(x, w):
    return pl.pallas_call(
        _mm_body,
        grid=(_NBLK,),
        in_specs=[pl.BlockSpec((_BLK, _D), lambda i: (i, 0)),
                  pl.BlockSpec((_D, _H), lambda i: (0, 0))],
        out_specs=pl.BlockSpec((_BLK, _H), lambda i: (i, 0)),
        out_shape=jax.ShapeDtypeStruct((_N, _H), jnp.float32),
    )(x, w)


def _scale_body(degp_ref, h_ref, hs_ref, dis_ref):
    deg = jnp.sum(degp_ref[...], axis=0)[:, :1] + 1.0
    dis = lax.rsqrt(deg)
    dis_ref[...] = dis
    hs_ref[...] = h_ref[...] * dis


def _scale(degp, h):
    return pl.pallas_call(
        _scale_body,
        grid=(_NBLK,),
        in_specs=[pl.BlockSpec((_NC, _BLK, _H), lambda i: (0, i, 0)),
                  pl.BlockSpec((_BLK, _H), lambda i: (i, 0))],
        out_specs=[pl.BlockSpec((_BLK, _H), lambda i: (i, 0)),
                   pl.BlockSpec((_BLK, 1), lambda i: (i, 0))],
        out_shape=[jax.ShapeDtypeStruct((_N, _H), jnp.float32),
                   jax.ShapeDtypeStruct((_N, 1), jnp.float32)],
    )(degp, h)


def _layer_body(p_ref, hs1_ref, dis_ref, b1_ref, w2_ref, o_ref):
    p = p_ref[...]
    agg = p[0] + p[1] + hs1_ref[...]
    h1 = jnp.maximum(agg * dis_ref[...] + b1_ref[...], 0.0)
    o_ref[...] = lax.dot_general(
        h1, w2_ref[...], (((1,), (0,)), ((), ())),
        preferred_element_type=jnp.float32,
        precision=lax.Precision.HIGHEST) * dis_ref[...]


def _layer(p, hs1, dis, b1, w2):
    return pl.pallas_call(
        _layer_body,
        grid=(_NBLK,),
        in_specs=[pl.BlockSpec((_NC, _BLK, _H), lambda i: (0, i, 0)),
                  pl.BlockSpec((_BLK, _H), lambda i: (i, 0)),
                  pl.BlockSpec((_BLK, 1), lambda i: (i, 0)),
                  pl.BlockSpec((1, _H), lambda i: (0, 0)),
                  pl.BlockSpec((_D, _H), lambda i: (0, 0))],
        out_specs=pl.BlockSpec((_BLK, _H), lambda i: (i, 0)),
        out_shape=jax.ShapeDtypeStruct((_N, _H), jnp.float32),
    )(p, hs1, dis, b1, w2)


def _final_body(q_ref, hs2_ref, dis_ref, b2_ref, batch_ref, gamma_ref,
                beta_ref, wm1_ref, bm1_ref, wm2_ref, bm2_ref, o_ref,
                msum, vsum, pool, cnt):
    i = pl.program_id(0)
    q = q_ref[...]
    h = (q[0] + q[1] + hs2_ref[...]) * dis_ref[...] + b2_ref[...]

    bvec = batch_ref[...].reshape(1, _BLK)
    gi = lax.broadcasted_iota(jnp.int32, (_G, _BLK), 0)
    oh = (bvec == gi).astype(jnp.float32)
    ps = lax.dot_general(oh, h, (((1,), (0,)), ((), ())),
                         preferred_element_type=jnp.float32,
                         precision=lax.Precision.HIGHEST)
    pc = jnp.sum(oh, axis=1, keepdims=True)
    ms = jnp.sum(h, axis=0, keepdims=True)
    vs = jnp.sum(h * h, axis=0, keepdims=True)

    @pl.when(i == 0)
    def _():
        msum[...] = ms
        vsum[...] = vs
        pool[...] = ps
        cnt[...] = pc

    @pl.when(i > 0)
    def _():
        msum[...] += ms
        vsum[...] += vs
        pool[...] += ps
        cnt[...] += pc

    @pl.when(i == _NBLK - 1)
    def _():
        mean = msum[...] * (1.0 / _N)
        var = vsum[...] * (1.0 / _N) - mean * mean
        scale = gamma_ref[...] * lax.rsqrt(var + 1e-5)
        pm = pool[...] / jnp.maximum(cnt[...], 1.0)
        pb = (pm - mean) * scale + beta_ref[...]
        z = jnp.maximum(
            lax.dot_general(pb, wm1_ref[...], (((1,), (0,)), ((), ())),
                            preferred_element_type=jnp.float32,
                            precision=lax.Precision.HIGHEST) + bm1_ref[...],
            0.0)
        o_ref[...] = lax.dot_general(
            z, wm2_ref[...], (((1,), (0,)), ((), ())),
            preferred_element_type=jnp.float32,
            precision=lax.Precision.HIGHEST) + bm2_ref[...]


def _final(q, hs2, dis, b2, batch3d, gamma, beta, wm1, bm1, wm2, bm2):
    return pl.pallas_call(
        _final_body,
        grid=(_NBLK,),
        in_specs=[pl.BlockSpec((_NC, _BLK, _H), lambda i: (0, i, 0)),
                  pl.BlockSpec((_BLK, _H), lambda i: (i, 0)),
                  pl.BlockSpec((_BLK, 1), lambda i: (i, 0)),
                  pl.BlockSpec((1, _H), lambda i: (0, 0)),
                  pl.BlockSpec((1, 1, _BLK), lambda i: (i, 0, 0)),
                  pl.BlockSpec((1, _H), lambda i: (0, 0)),
                  pl.BlockSpec((1, _H), lambda i: (0, 0)),
                  pl.BlockSpec((_H, _H), lambda i: (0, 0)),
                  pl.BlockSpec((1, _H), lambda i: (0, 0)),
                  pl.BlockSpec((_H, 1), lambda i: (0, 0)),
                  pl.BlockSpec((1, 1), lambda i: (0, 0))],
        out_specs=pl.BlockSpec((_G, 1), lambda i: (0, 0)),
        out_shape=jax.ShapeDtypeStruct((_G, 1), jnp.float32),
        scratch_shapes=[pltpu.VMEM((1, _H), jnp.float32),
                        pltpu.VMEM((1, _H), jnp.float32),
                        pltpu.VMEM((_G, _H), jnp.float32),
                        pltpu.VMEM((_G, 1), jnp.float32)],
    )(q, hs2, dis, b2, batch3d, gamma, beta, wm1, bm1, wm2, bm2)


def kernel(x, edge_index, batch, W1, b1, W2, b2, gamma, beta, Wm1, bm1,
           Wm2, bm2):
    src = edge_index[0]
    dst = edge_index[1]
    pad = _EPAD - _E
    src2d = jnp.concatenate(
        [src, jnp.zeros((pad,), jnp.int32)]).reshape(_EPAD // _CHUNK, _CHUNK)
    dst2d = jnp.concatenate(
        [dst, jnp.full((pad,), _N, jnp.int32)]).reshape(_EPAD // _CHUNK, _CHUNK)

    degp = _deg(dst2d, jnp.ones((_CHUNK, _H), jnp.float32))
    h1pre = _mm(x, W1)
    hs1, dis = _scale(degp, h1pre)

    p = _agg(hs1, src2d, dst2d)
    hs2 = _layer(p, hs1, dis, b1.reshape(1, _H), W2)
    q = _agg(hs2, src2d, dst2d)

    return _final(q, hs2, dis, b2.reshape(1, _H),
                  batch.reshape(_NBLK, 1, _BLK), gamma.reshape(1, _H),
                  beta.reshape(1, _H), Wm1, bm1.reshape(1, _H), Wm2,
                  bm2.reshape(1, 1))
